# baseline (device time: 382163 ns/iter reference)
import jax
import jax.numpy as jnp
from jax import lax
from jax.experimental import pallas as pl
from jax.experimental.pallas import tpu as pltpu

N_DEV = 16
NSLOTS = 3
NSUB = 8
LAST = N_DEV - 2


def kernel(x, w_mat):
    m, k_shard = x.shape
    _, n_out = w_mat.shape
    chunk = m // N_DEV
    subw = n_out // NSUB

    def rows(c):
        return pl.ds(c * chunk, chunk)

    def cols(r):
        return pl.ds(r * subw, subw)

    def body(x_ref, w_ref, out_hbm, acc, comm,
             rs_send, rs_recv, ag_send, ag_recv, credit_rs, credit_ag,
             copy_sem, copy_own_sem):
        me = lax.axis_index("i")
        left = lax.rem(me + (N_DEV - 1), N_DEV)
        right = lax.rem(me + 1, N_DEV)

        barrier_sem = pltpu.get_barrier_semaphore()
        for nbr in (left, right):
            pl.semaphore_signal(
                barrier_sem, inc=1,
                device_id=(nbr,), device_id_type=pl.DeviceIdType.MESH,
            )
        pl.semaphore_wait(barrier_sem, 2)

        def gemm_chunk(c):
            rc = rows(c)
            acc[rc, :] = jnp.dot(
                x_ref[rc, :], w_ref[...], preferred_element_type=jnp.float32
            )

        gemm_chunk(me)
        gemm_chunk(lax.rem(me + 1, N_DEV))
        gemm_chunk(lax.rem(me + (N_DEV - 1), N_DEV))

        def dst_dev(r):
            return right if r < NSUB // 2 else left

        def src_dev(r):
            return left if r < NSUB // 2 else right

        def rs_send_chunk(r, s):
            return lax.rem(me + (N_DEV - s if r < NSUB // 2 else s), N_DEV)

        def rs_recv_chunk(r, s):
            return lax.rem(me + (N_DEV - s - 1 if r < NSUB // 2 else s + 1), N_DEV)

        def rs_desc(r, s):
            slot = s % NSLOTS
            return pltpu.make_async_remote_copy(
                src_ref=acc.at[rows(rs_send_chunk(r, s)), cols(r)],
                dst_ref=comm.at[r, slot],
                send_sem=rs_send.at[r, slot],
                recv_sem=rs_recv.at[r, slot],
                device_id=(dst_dev(r),),
                device_id_type=pl.DeviceIdType.MESH,
            )

        for r in range(NSUB):
            rs_desc(r, 0).start()
        for s in range(N_DEV - 1):
            for r in range(NSUB):
                rs_desc(r, s).wait_recv()
                rc = rows(rs_recv_chunk(r, s))
                acc[rc, cols(r)] = acc[rc, cols(r)] + comm[r, s % NSLOTS]
                if s + NSLOTS <= LAST:
                    pl.semaphore_signal(
                        credit_rs.at[r], inc=1,
                        device_id=(src_dev(r),),
                        device_id_type=pl.DeviceIdType.MESH,
                    )
                if s + 1 <= LAST:
                    if s + 1 >= NSLOTS:
                        pl.semaphore_wait(credit_rs.at[r], 1)
                        rs_desc(r, s + 1 - NSLOTS).wait_send()
                    rs_desc(r, s + 1).start()
            if s + 2 <= N_DEV // 2:
                gemm_chunk(lax.rem(me + (N_DEV - s - 2), N_DEV))
                if s + 2 < N_DEV // 2:
                    gemm_chunk(lax.rem(me + s + 2, N_DEV))
        for r in range(NSUB):
            for k in range(min(NSLOTS, N_DEV - 1)):
                rs_desc(r, LAST + 1 - k).wait_send()

        for r in range(NSUB):
            co = lax.rem(me + (1 if r < NSUB // 2 else N_DEV - 1), N_DEV)
            ro = rows(co)
            acc[ro, cols(r)] = jnp.maximum(acc[ro, cols(r)], 0.0)
            pltpu.make_async_copy(
                acc.at[ro, cols(r)], out_hbm.at[ro, cols(r)],
                copy_own_sem.at[r],
            ).start()

        def ag_chunk(r, s):
            return lax.rem(me + (N_DEV + 1 - s if r < NSUB // 2 else N_DEV - 1 + s), N_DEV)

        def ag_desc(r, s):
            slot = s % NSLOTS
            target = acc.at[rows(ag_chunk(r, s)), cols(r)]
            return pltpu.make_async_remote_copy(
                src_ref=target,
                dst_ref=target,
                send_sem=ag_send.at[r, slot],
                recv_sem=ag_recv.at[r, slot],
                device_id=(dst_dev(r),),
                device_id_type=pl.DeviceIdType.MESH,
            )

        def hbm_copy(r, s):
            rc = rows(lax.rem(me + (N_DEV - s if r < NSUB // 2 else s), N_DEV))
            return pltpu.make_async_copy(
                acc.at[rc, cols(r)], out_hbm.at[rc, cols(r)],
                copy_sem.at[r, s % NSLOTS],
            )

        for r in range(NSUB):
            ag_desc(r, 0).start()
        for s in range(N_DEV - 1):
            for r in range(NSUB):
                ag_desc(r, s).wait_recv()
                if s >= NSLOTS:
                    hbm_copy(r, s - NSLOTS).wait()
                hbm_copy(r, s).start()
                if s + NSLOTS <= LAST:
                    pl.semaphore_signal(
                        credit_ag.at[r], inc=1,
                        device_id=(src_dev(r),),
                        device_id_type=pl.DeviceIdType.MESH,
                    )
                if s + 1 <= LAST:
                    if s + 1 >= NSLOTS:
                        pl.semaphore_wait(credit_ag.at[r], 1)
                        ag_desc(r, s + 1 - NSLOTS).wait_send()
                    ag_desc(r, s + 1).start()
        for r in range(NSUB):
            for k in range(min(NSLOTS, N_DEV - 1)):
                ag_desc(r, LAST + 1 - k).wait_send()
                hbm_copy(r, LAST + 1 - k).wait()
            pltpu.make_async_copy(
                acc.at[rows(0), cols(r)], out_hbm.at[rows(0), cols(r)],
                copy_own_sem.at[r],
            ).wait()

    return pl.pallas_call(
        body,
        out_shape=jax.ShapeDtypeStruct((m, n_out), jnp.float32),
        in_specs=[
            pl.BlockSpec(memory_space=pltpu.VMEM),
            pl.BlockSpec(memory_space=pltpu.VMEM),
        ],
        out_specs=pl.BlockSpec(memory_space=pltpu.MemorySpace.HBM),
        scratch_shapes=[
            pltpu.VMEM((m, n_out), jnp.float32),
            pltpu.VMEM((NSUB, NSLOTS, chunk, subw), jnp.float32),
            pltpu.SemaphoreType.DMA((NSUB, NSLOTS)),
            pltpu.SemaphoreType.DMA((NSUB, NSLOTS)),
            pltpu.SemaphoreType.DMA((NSUB, NSLOTS)),
            pltpu.SemaphoreType.DMA((NSUB, NSLOTS)),
            pltpu.SemaphoreType.REGULAR((NSUB,)),
            pltpu.SemaphoreType.REGULAR((NSUB,)),
            pltpu.SemaphoreType.DMA((NSUB, NSLOTS)),
            pltpu.SemaphoreType.DMA((NSUB,)),
        ],
        compiler_params=pltpu.CompilerParams(
            collective_id=0,
            vmem_limit_bytes=56 * 1024 * 1024,
        ),
    )(x, w_mat)


# device time: 380845 ns/iter; 1.0035x vs baseline; 1.0035x over previous
import jax
import jax.numpy as jnp
from jax import lax
from jax.experimental import pallas as pl
from jax.experimental.pallas import tpu as pltpu

N_DEV = 16
NSLOTS = 2
NSUB = 4
LAST = N_DEV - 2


def kernel(x, w_mat):
    m, k_shard = x.shape
    _, n_out = w_mat.shape
    chunk = m // N_DEV
    subw = n_out // NSUB

    def rows(c):
        return pl.ds(c * chunk, chunk)

    def cols(r):
        return pl.ds(r * subw, subw)

    def body(x_ref, w_ref, out_hbm, acc, comm,
             rs_send, rs_recv, ag_send, ag_recv, credit_rs, credit_ag,
             copy_sem, copy_own_sem):
        me = lax.axis_index("i")
        left = lax.rem(me + (N_DEV - 1), N_DEV)
        right = lax.rem(me + 1, N_DEV)

        barrier_sem = pltpu.get_barrier_semaphore()
        for nbr in (left, right):
            pl.semaphore_signal(
                barrier_sem, inc=1,
                device_id=(nbr,), device_id_type=pl.DeviceIdType.MESH,
            )
        pl.semaphore_wait(barrier_sem, 2)

        def gemm_chunk(c):
            rc = rows(c)
            acc[rc, :] = jnp.dot(
                x_ref[rc, :], w_ref[...], preferred_element_type=jnp.float32
            )

        gemm_chunk(me)
        gemm_chunk(lax.rem(me + 1, N_DEV))
        gemm_chunk(lax.rem(me + (N_DEV - 1), N_DEV))

        def dst_dev(r):
            return right if r < NSUB // 2 else left

        def src_dev(r):
            return left if r < NSUB // 2 else right

        def rs_send_chunk(r, s):
            return lax.rem(me + (N_DEV - s if r < NSUB // 2 else s), N_DEV)

        def rs_recv_chunk(r, s):
            return lax.rem(me + (N_DEV - s - 1 if r < NSUB // 2 else s + 1), N_DEV)

        def rs_desc(r, s):
            slot = s % NSLOTS
            return pltpu.make_async_remote_copy(
                src_ref=acc.at[rows(rs_send_chunk(r, s)), cols(r)],
                dst_ref=comm.at[r, slot],
                send_sem=rs_send.at[r, slot],
                recv_sem=rs_recv.at[r, slot],
                device_id=(dst_dev(r),),
                device_id_type=pl.DeviceIdType.MESH,
            )

        for r in range(NSUB):
            rs_desc(r, 0).start()
        for s in range(N_DEV - 1):
            for r in range(NSUB):
                rs_desc(r, s).wait_recv()
                rc = rows(rs_recv_chunk(r, s))
                acc[rc, cols(r)] = acc[rc, cols(r)] + comm[r, s % NSLOTS]
                if s + NSLOTS <= LAST:
                    pl.semaphore_signal(
                        credit_rs.at[r], inc=1,
                        device_id=(src_dev(r),),
                        device_id_type=pl.DeviceIdType.MESH,
                    )
                if s + 1 <= LAST:
                    if s + 1 >= NSLOTS:
                        pl.semaphore_wait(credit_rs.at[r], 1)
                        rs_desc(r, s + 1 - NSLOTS).wait_send()
                    rs_desc(r, s + 1).start()
            if s + 2 <= N_DEV // 2:
                gemm_chunk(lax.rem(me + (N_DEV - s - 2), N_DEV))
                if s + 2 < N_DEV // 2:
                    gemm_chunk(lax.rem(me + s + 2, N_DEV))
        for r in range(NSUB):
            for k in range(min(NSLOTS, N_DEV - 1)):
                rs_desc(r, LAST + 1 - k).wait_send()

        for r in range(NSUB):
            co = lax.rem(me + (1 if r < NSUB // 2 else N_DEV - 1), N_DEV)
            ro = rows(co)
            acc[ro, cols(r)] = jnp.maximum(acc[ro, cols(r)], 0.0)
            pltpu.make_async_copy(
                acc.at[ro, cols(r)], out_hbm.at[ro, cols(r)],
                copy_own_sem.at[r],
            ).start()

        def ag_chunk(r, s):
            return lax.rem(me + (N_DEV + 1 - s if r < NSUB // 2 else N_DEV - 1 + s), N_DEV)

        def ag_desc(r, s):
            slot = s % NSLOTS
            target = acc.at[rows(ag_chunk(r, s)), cols(r)]
            return pltpu.make_async_remote_copy(
                src_ref=target,
                dst_ref=target,
                send_sem=ag_send.at[r, slot],
                recv_sem=ag_recv.at[r, slot],
                device_id=(dst_dev(r),),
                device_id_type=pl.DeviceIdType.MESH,
            )

        def hbm_copy(r, s):
            rc = rows(lax.rem(me + (N_DEV - s if r < NSUB // 2 else s), N_DEV))
            return pltpu.make_async_copy(
                acc.at[rc, cols(r)], out_hbm.at[rc, cols(r)],
                copy_sem.at[r, s % NSLOTS],
            )

        for r in range(NSUB):
            ag_desc(r, 0).start()
        for s in range(N_DEV - 1):
            for r in range(NSUB):
                ag_desc(r, s).wait_recv()
                if s >= NSLOTS:
                    hbm_copy(r, s - NSLOTS).wait()
                hbm_copy(r, s).start()
                if s + NSLOTS <= LAST:
                    pl.semaphore_signal(
                        credit_ag.at[r], inc=1,
                        device_id=(src_dev(r),),
                        device_id_type=pl.DeviceIdType.MESH,
                    )
                if s + 1 <= LAST:
                    if s + 1 >= NSLOTS:
                        pl.semaphore_wait(credit_ag.at[r], 1)
                        ag_desc(r, s + 1 - NSLOTS).wait_send()
                    ag_desc(r, s + 1).start()
        for r in range(NSUB):
            for k in range(min(NSLOTS, N_DEV - 1)):
                ag_desc(r, LAST + 1 - k).wait_send()
                hbm_copy(r, LAST + 1 - k).wait()
            pltpu.make_async_copy(
                acc.at[rows(0), cols(r)], out_hbm.at[rows(0), cols(r)],
                copy_own_sem.at[r],
            ).wait()

    return pl.pallas_call(
        body,
        out_shape=jax.ShapeDtypeStruct((m, n_out), jnp.float32),
        in_specs=[
            pl.BlockSpec(memory_space=pltpu.VMEM),
            pl.BlockSpec(memory_space=pltpu.VMEM),
        ],
        out_specs=pl.BlockSpec(memory_space=pltpu.MemorySpace.HBM),
        scratch_shapes=[
            pltpu.VMEM((m, n_out), jnp.float32),
            pltpu.VMEM((NSUB, NSLOTS, chunk, subw), jnp.float32),
            pltpu.SemaphoreType.DMA((NSUB, NSLOTS)),
            pltpu.SemaphoreType.DMA((NSUB, NSLOTS)),
            pltpu.SemaphoreType.DMA((NSUB, NSLOTS)),
            pltpu.SemaphoreType.DMA((NSUB, NSLOTS)),
            pltpu.SemaphoreType.REGULAR((NSUB,)),
            pltpu.SemaphoreType.REGULAR((NSUB,)),
            pltpu.SemaphoreType.DMA((NSUB, NSLOTS)),
            pltpu.SemaphoreType.DMA((NSUB,)),
        ],
        compiler_params=pltpu.CompilerParams(
            collective_id=0,
            vmem_limit_bytes=56 * 1024 * 1024,
        ),
    )(x, w_mat)
